# baseline (speedup 1.0000x reference)
import jax
import jax.numpy as jnp
from jax import lax
from jax.experimental import pallas as pl
from jax.experimental.pallas import tpu as pltpu
from jax.experimental.pallas import tpu_sc as plsc

N_NODES = 10000
D_FEAT = 128
N_EDGES = 320000
D_PACK = D_FEAT // 2

NC = 2
NS = 16
NW = NC * NS
E_PER_W = N_EDGES // NW
CHUNK = 80
NCHUNK = E_PER_W // CHUNK
GROUPS = CHUNK // 16
LANES = 16
SLAB = 624
SLAB_LAST = N_NODES - (NS - 1) * SLAB
HI_MASK = -65536


STAGE_ROWS = 125
STAGE_BLOCKS = N_NODES // NS // STAGE_ROWS


def _sc_body(h_hbm, src_hbm, dst_hbm, out_hbm,
             src_v, dst_v, hu0, hv0, hu1, hv1, out_v, stage_v, h_sh,
             sem_u0, sem_v0, sem_u1, sem_v1):
    cid = lax.axis_index("c")
    sid = lax.axis_index("s")
    wid = sid * NC + cid

    def stage_body(b, carry):
        start = (sid * STAGE_BLOCKS + b) * STAGE_ROWS
        pltpu.sync_copy(h_hbm.at[pl.ds(start, STAGE_ROWS)], stage_v)
        pltpu.sync_copy(stage_v, h_sh.at[pl.ds(start, STAGE_ROWS)])
        return carry

    lax.fori_loop(0, STAGE_BLOCKS, stage_body, 0, unroll=False)

    pltpu.sync_copy(src_hbm.at[wid], src_v)
    pltpu.sync_copy(dst_hbm.at[wid], dst_v)

    plsc.subcore_barrier()

    bufs = ((hu0, hv0, sem_u0, sem_v0), (hu1, hv1, sem_u1, sem_v1))

    def issue(ci, b):
        hu, hv, su, sv = bufs[b]
        pltpu.async_copy(h_sh.at[src_v.at[ci]], hu, su)
        pltpu.async_copy(h_sh.at[dst_v.at[ci]], hv, sv)

    def wait(b):
        hu, hv, su, sv = bufs[b]
        pltpu.make_async_copy(h_sh.at[src_v.at[0]], hu, su).wait()
        pltpu.make_async_copy(h_sh.at[dst_v.at[0]], hv, sv).wait()

    def compute(ci, b):
        hu, hv = bufs[b][0], bufs[b][1]

        def group_body(g, carry):
            eids = g * LANES + lax.iota(jnp.int32, LANES)
            zero = jnp.zeros((LANES,), jnp.float32)

            @plsc.parallel_loop(0, D_PACK, step=4, unroll=2,
                                carry=(zero, zero, zero, zero))
            def dloop(d, accs):
                dv = jnp.broadcast_to(d, (LANES,))
                new = []
                for k in range(4):
                    pu = plsc.load_gather(hu, [eids, dv + k])
                    pv = plsc.load_gather(hv, [eids, dv + k])
                    ulo = plsc.bitcast(lax.shift_left(pu, 16), jnp.float32)
                    vlo = plsc.bitcast(lax.shift_left(pv, 16), jnp.float32)
                    uhi = plsc.bitcast(pu & HI_MASK, jnp.float32)
                    vhi = plsc.bitcast(pv & HI_MASK, jnp.float32)
                    new.append(accs[k] + ulo * vlo + uhi * vhi)
                return tuple(new)

            a = dloop
            out_v[pl.ds(ci * CHUNK + g * LANES, LANES)] = (a[0] + a[1]) + (a[2] + a[3])
            return carry

        lax.fori_loop(0, GROUPS, group_body, 0, unroll=False)

    issue(0, 0)

    def pair_body(p, carry):
        c0 = 2 * p
        issue(c0 + 1, 1)
        wait(0)
        compute(c0, 0)
        issue(c0 + 2, 0)
        wait(1)
        compute(c0 + 1, 1)
        return carry

    lax.fori_loop(0, (NCHUNK - 1) // 2, pair_body, 0, unroll=False)
    wait(0)
    compute(NCHUNK - 1, 0)

    pltpu.sync_copy(out_v, out_hbm.at[wid])


@jax.jit
def kernel(h, edge_index):
    hp = lax.bitcast_convert_type(
        h.astype(jnp.bfloat16).reshape(N_NODES, D_PACK, 2), jnp.int32)
    ei = edge_index.astype(jnp.int32)
    src = ei[0].reshape(NW, NCHUNK, CHUNK)
    dst = ei[1].reshape(NW, NCHUNK, CHUNK)
    mesh = plsc.VectorSubcoreMesh(core_axis_name="c", subcore_axis_name="s")
    run = pl.kernel(
        _sc_body,
        out_type=jax.ShapeDtypeStruct((NW, E_PER_W), jnp.float32),
        mesh=mesh,
        compiler_params=pltpu.CompilerParams(needs_layout_passes=False,
                                             use_tc_tiling_on_sc=False),
        scratch_types=[
            pltpu.VMEM((NCHUNK, CHUNK), jnp.int32),
            pltpu.VMEM((NCHUNK, CHUNK), jnp.int32),
            pltpu.VMEM((CHUNK, D_PACK), jnp.int32),
            pltpu.VMEM((CHUNK, D_PACK), jnp.int32),
            pltpu.VMEM((CHUNK, D_PACK), jnp.int32),
            pltpu.VMEM((CHUNK, D_PACK), jnp.int32),
            pltpu.VMEM((E_PER_W,), jnp.float32),
            pltpu.VMEM((STAGE_ROWS, D_PACK), jnp.int32),
            pltpu.VMEM_SHARED((N_NODES, D_PACK), jnp.int32),
            pltpu.SemaphoreType.DMA,
            pltpu.SemaphoreType.DMA,
            pltpu.SemaphoreType.DMA,
            pltpu.SemaphoreType.DMA,
        ],
    )
    out = run(hp, src, dst)
    return out.reshape(N_EDGES, 1)


# speedup vs baseline: 2.1505x; 2.1505x over previous
import jax
import jax.numpy as jnp
from jax import lax
from jax.experimental import pallas as pl
from jax.experimental.pallas import tpu as pltpu
from jax.experimental.pallas import tpu_sc as plsc

N_NODES = 10000
D_FEAT = 128
N_EDGES = 320000
D_PACK = D_FEAT // 2

NC = 2
NS = 16
FS = 8
PW = D_PACK // FS
NSHARD = NS // FS
E_PER_SC = N_EDGES // NC
E_PER_SHARD = E_PER_SC // NSHARD
BATCH_E = 2000
NBATCH = E_PER_SHARD // BATCH_E
NPAIR = NBATCH // 2
NGROUP = BATCH_E // 16
NROWS = NSHARD * NBATCH
ROWS_PER_TILE = NROWS // NS
LANES = 16
HI_MASK = -65536


def _sc_body(hpt_hbm, src_hbm, dst_hbm, rid_hbm, out_hbm,
             table_v, sb0, db0, sb1, db1, pb0, pb1, rid_v, stage_v, acc_sh,
             sem_s0, sem_d0, sem_s1, sem_d1, sem_a0, sem_a1):
    cid = lax.axis_index("c")
    sid = lax.axis_index("s")
    fs = sid % FS
    shard = sid // FS

    pltpu.sync_copy(hpt_hbm.at[fs], table_v)
    pltpu.sync_copy(rid_hbm, rid_v)

    def zrow(r, carry):
        def zcol(i, carry2):
            stage_v[r, pl.ds(i * LANES, LANES)] = jnp.zeros((LANES,), jnp.float32)
            return carry2
        lax.fori_loop(0, BATCH_E // LANES, zcol, 0, unroll=False)
        return carry
    lax.fori_loop(0, ROWS_PER_TILE, zrow, 0, unroll=False)
    pltpu.sync_copy(stage_v, acc_sh.at[pl.ds(sid * ROWS_PER_TILE, ROWS_PER_TILE)])

    plsc.subcore_barrier()

    ibufs = ((sb0, db0, sem_s0, sem_d0), (sb1, db1, sem_s1, sem_d1))
    pbufs = ((pb0, sem_a0), (pb1, sem_a1))

    def issue_idx(b, k):
        sb, db, ss, sd = ibufs[k]
        pltpu.async_copy(src_hbm.at[cid, shard, b], sb, ss)
        pltpu.async_copy(dst_hbm.at[cid, shard, b], db, sd)

    def wait_idx(k):
        sb, db, ss, sd = ibufs[k]
        pltpu.make_async_copy(src_hbm.at[0, 0, 0], sb, ss).wait()
        pltpu.make_async_copy(dst_hbm.at[0, 0, 0], db, sd).wait()

    def issue_add(b, k):
        pb, sa = pbufs[k]
        pltpu.async_copy(pb, acc_sh.at[rid_v.at[shard * NBATCH + b]], sa, add=True)

    def wait_add(k):
        pb, sa = pbufs[k]
        pltpu.make_async_copy(pb, acc_sh.at[rid_v.at[0]], sa).wait()

    def compute(k):
        sb, db = ibufs[k][0], ibufs[k][1]
        pb = pbufs[k][0]

        @plsc.parallel_loop(0, NGROUP, step=1, unroll=5)
        def _(g):
            su = sb[pl.ds(g * LANES, LANES)]
            dv = db[pl.ds(g * LANES, LANES)]
            ub = lax.shift_left(su, 3)
            vb = lax.shift_left(dv, 3)
            acc0 = jnp.zeros((LANES,), jnp.float32)
            acc1 = jnp.zeros((LANES,), jnp.float32)
            for kk in range(PW):
                pu = plsc.load_gather(table_v, [ub + kk])
                pv = plsc.load_gather(table_v, [vb + kk])
                ulo = plsc.bitcast(lax.shift_left(pu, 16), jnp.float32)
                vlo = plsc.bitcast(lax.shift_left(pv, 16), jnp.float32)
                uhi = plsc.bitcast(pu & HI_MASK, jnp.float32)
                vhi = plsc.bitcast(pv & HI_MASK, jnp.float32)
                acc0 = acc0 + ulo * vlo
                acc1 = acc1 + uhi * vhi
            pb[0, pl.ds(g * LANES, LANES)] = acc0 + acc1

    issue_idx(0, 0)

    def pair_body(p, carry):
        b0 = 2 * p
        issue_idx(b0 + 1, 1)
        wait_idx(0)

        @pl.when(p >= 1)
        def _():
            wait_add(0)

        compute(0)
        issue_add(b0, 0)

        @pl.when(p <= NPAIR - 2)
        def _():
            issue_idx(b0 + 2, 0)

        wait_idx(1)

        @pl.when(p >= 1)
        def _():
            wait_add(1)

        compute(1)
        issue_add(b0 + 1, 1)
        return carry

    lax.fori_loop(0, NPAIR, pair_body, 0, unroll=False)
    wait_add(0)
    wait_add(1)

    plsc.subcore_barrier()

    pltpu.sync_copy(acc_sh.at[pl.ds(sid * ROWS_PER_TILE, ROWS_PER_TILE)], stage_v)
    pltpu.sync_copy(stage_v, out_hbm.at[cid, pl.ds(sid * ROWS_PER_TILE, ROWS_PER_TILE)])


@jax.jit
def kernel(h, edge_index):
    hp = lax.bitcast_convert_type(
        h.astype(jnp.bfloat16).reshape(N_NODES, D_PACK, 2), jnp.int32)
    hpt = hp.reshape(N_NODES, FS, PW).transpose(1, 0, 2).reshape(FS, N_NODES * PW)
    ei = edge_index.astype(jnp.int32)
    src = ei[0].reshape(NC, NSHARD, NBATCH, BATCH_E)
    dst = ei[1].reshape(NC, NSHARD, NBATCH, BATCH_E)
    rowids = jnp.arange(NROWS, dtype=jnp.int32).reshape(NROWS, 1)
    mesh = plsc.VectorSubcoreMesh(core_axis_name="c", subcore_axis_name="s")
    run = pl.kernel(
        _sc_body,
        out_type=jax.ShapeDtypeStruct((NC, NROWS, BATCH_E), jnp.float32),
        mesh=mesh,
        compiler_params=pltpu.CompilerParams(needs_layout_passes=False,
                                             use_tc_tiling_on_sc=False),
        scratch_types=[
            pltpu.VMEM((N_NODES * PW,), jnp.int32),
            pltpu.VMEM((BATCH_E,), jnp.int32),
            pltpu.VMEM((BATCH_E,), jnp.int32),
            pltpu.VMEM((BATCH_E,), jnp.int32),
            pltpu.VMEM((BATCH_E,), jnp.int32),
            pltpu.VMEM((1, BATCH_E), jnp.float32),
            pltpu.VMEM((1, BATCH_E), jnp.float32),
            pltpu.VMEM((NROWS, 1), jnp.int32),
            pltpu.VMEM((ROWS_PER_TILE, BATCH_E), jnp.float32),
            pltpu.VMEM_SHARED((NROWS, BATCH_E), jnp.float32),
            pltpu.SemaphoreType.DMA,
            pltpu.SemaphoreType.DMA,
            pltpu.SemaphoreType.DMA,
            pltpu.SemaphoreType.DMA,
            pltpu.SemaphoreType.DMA,
            pltpu.SemaphoreType.DMA,
        ],
    )
    out = run(hpt, src, dst, rowids)
    return out.reshape(N_EDGES, 1)


# speedup vs baseline: 4.3685x; 2.0314x over previous
import jax
import jax.numpy as jnp
from jax import lax
from jax.experimental import pallas as pl
from jax.experimental.pallas import tpu as pltpu
from jax.experimental.pallas import tpu_sc as plsc

N_NODES = 10000
D_FEAT = 128
N_EDGES = 320000
D_PACK = D_FEAT // 2

NC = 2
NS = 16
FS = 8
PW = D_PACK // FS
NSHARD = NS // FS
E_PER_SC = N_EDGES // NC
E_PER_SHARD = E_PER_SC // NSHARD
BATCH_E = 4000
ROW_E = 2000
RPB = BATCH_E // ROW_E
NBATCH = E_PER_SHARD // BATCH_E
NPAIR = NBATCH // 2
NGROUP = ROW_E // 16
NROWS = NSHARD * NBATCH * RPB
ROWS_PER_TILE = NROWS // NS
LANES = 16
HI_MASK = -65536


def _sc_body(hpt_hbm, src_hbm, dst_hbm, rid_hbm, out_hbm,
             t0, t1, t2, t3, t4, t5, t6, t7,
             sb0, db0, sb1, db1, pb0, pb1, rid_v, stage_v, acc_sh,
             sem_s0, sem_d0, sem_s1, sem_d1, sem_a0, sem_a1):
    cid = lax.axis_index("c")
    sid = lax.axis_index("s")
    fs = sid % FS
    shard = sid // FS
    tabs = (t0, t1, t2, t3, t4, t5, t6, t7)

    for kk in range(PW):
        pltpu.sync_copy(hpt_hbm.at[fs, kk], tabs[kk])
    pltpu.sync_copy(rid_hbm, rid_v)

    def zrow(r, carry):
        def zcol(i, carry2):
            stage_v[r, pl.ds(i * LANES, LANES)] = jnp.zeros((LANES,), jnp.float32)
            return carry2
        lax.fori_loop(0, ROW_E // LANES, zcol, 0, unroll=False)
        return carry
    lax.fori_loop(0, ROWS_PER_TILE, zrow, 0, unroll=False)
    pltpu.sync_copy(stage_v, acc_sh.at[pl.ds(sid * ROWS_PER_TILE, ROWS_PER_TILE)])

    plsc.subcore_barrier()

    ibufs = ((sb0, db0, sem_s0, sem_d0), (sb1, db1, sem_s1, sem_d1))
    pbufs = ((pb0, sem_a0), (pb1, sem_a1))

    def issue_idx(b, k):
        sb, db, ss, sd = ibufs[k]
        pltpu.async_copy(src_hbm.at[cid, shard, b], sb, ss)
        pltpu.async_copy(dst_hbm.at[cid, shard, b], db, sd)

    def wait_idx(k):
        sb, db, ss, sd = ibufs[k]
        pltpu.make_async_copy(src_hbm.at[0, 0, 0], sb, ss).wait()
        pltpu.make_async_copy(dst_hbm.at[0, 0, 0], db, sd).wait()

    def issue_add(b, k):
        pb, sa = pbufs[k]
        pltpu.async_copy(pb, acc_sh.at[rid_v.at[shard * NBATCH + b]], sa, add=True)

    def wait_add(k):
        pb, sa = pbufs[k]
        pltpu.make_async_copy(pb, acc_sh.at[rid_v.at[0]], sa).wait()

    def compute(k):
        sb, db = ibufs[k][0], ibufs[k][1]
        pb = pbufs[k][0]

        for r in range(RPB):
            @plsc.parallel_loop(0, NGROUP, step=1, unroll=5)
            def _(g):
                su = sb[pl.ds(r * ROW_E + g * LANES, LANES)]
                dv = db[pl.ds(r * ROW_E + g * LANES, LANES)]
                acc0 = jnp.zeros((LANES,), jnp.float32)
                acc1 = jnp.zeros((LANES,), jnp.float32)
                for kk in range(PW):
                    pu = plsc.load_gather(tabs[kk], [su])
                    pv = plsc.load_gather(tabs[kk], [dv])
                    ulo = plsc.bitcast(lax.shift_left(pu, 16), jnp.float32)
                    vlo = plsc.bitcast(lax.shift_left(pv, 16), jnp.float32)
                    uhi = plsc.bitcast(pu & HI_MASK, jnp.float32)
                    vhi = plsc.bitcast(pv, jnp.float32)
                    acc0 = acc0 + ulo * vlo
                    acc1 = acc1 + uhi * vhi
                pb[r, pl.ds(g * LANES, LANES)] = acc0 + acc1

    issue_idx(0, 0)

    def pair_body(p, carry):
        b0 = 2 * p
        issue_idx(b0 + 1, 1)
        wait_idx(0)

        @pl.when(p >= 1)
        def _():
            wait_add(0)

        compute(0)
        issue_add(b0, 0)

        @pl.when(p <= NPAIR - 2)
        def _():
            issue_idx(b0 + 2, 0)

        wait_idx(1)

        @pl.when(p >= 1)
        def _():
            wait_add(1)

        compute(1)
        issue_add(b0 + 1, 1)
        return carry

    lax.fori_loop(0, NPAIR, pair_body, 0, unroll=False)
    wait_add(0)
    wait_add(1)

    plsc.subcore_barrier()

    pltpu.sync_copy(acc_sh.at[pl.ds(sid * ROWS_PER_TILE, ROWS_PER_TILE)], stage_v)
    pltpu.sync_copy(stage_v, out_hbm.at[cid, pl.ds(sid * ROWS_PER_TILE, ROWS_PER_TILE)])


@jax.jit
def kernel(h, edge_index):
    hp = lax.bitcast_convert_type(
        h.astype(jnp.bfloat16).reshape(N_NODES, D_PACK, 2), jnp.int32)
    hpt = hp.reshape(N_NODES, FS, PW).transpose(1, 2, 0)
    ei = edge_index.astype(jnp.int32)
    src = ei[0].reshape(NC, NSHARD, NBATCH, BATCH_E)
    dst = ei[1].reshape(NC, NSHARD, NBATCH, BATCH_E)
    rowids = jnp.arange(NROWS, dtype=jnp.int32).reshape(NSHARD * NBATCH, RPB)
    mesh = plsc.VectorSubcoreMesh(core_axis_name="c", subcore_axis_name="s")
    run = pl.kernel(
        _sc_body,
        out_type=jax.ShapeDtypeStruct((NC, NROWS, ROW_E), jnp.float32),
        mesh=mesh,
        compiler_params=pltpu.CompilerParams(needs_layout_passes=False,
                                             use_tc_tiling_on_sc=False),
        scratch_types=(
            [pltpu.VMEM((N_NODES,), jnp.int32) for _ in range(PW)] +
            [
                pltpu.VMEM((BATCH_E,), jnp.int32),
                pltpu.VMEM((BATCH_E,), jnp.int32),
                pltpu.VMEM((BATCH_E,), jnp.int32),
                pltpu.VMEM((BATCH_E,), jnp.int32),
                pltpu.VMEM((RPB, ROW_E), jnp.float32),
                pltpu.VMEM((RPB, ROW_E), jnp.float32),
                pltpu.VMEM((NSHARD * NBATCH, RPB), jnp.int32),
                pltpu.VMEM((ROWS_PER_TILE, ROW_E), jnp.float32),
                pltpu.VMEM_SHARED((NROWS, ROW_E), jnp.float32),
                pltpu.SemaphoreType.DMA,
                pltpu.SemaphoreType.DMA,
                pltpu.SemaphoreType.DMA,
                pltpu.SemaphoreType.DMA,
                pltpu.SemaphoreType.DMA,
                pltpu.SemaphoreType.DMA,
            ]
        ),
    )
    out = run(hpt, src, dst, rowids)
    return out.reshape(N_EDGES, 1)


# speedup vs baseline: 4.5386x; 1.0390x over previous
import jax
import jax.numpy as jnp
from jax import lax
from jax.experimental import pallas as pl
from jax.experimental.pallas import tpu as pltpu
from jax.experimental.pallas import tpu_sc as plsc

N_NODES = 10000
D_FEAT = 128
N_EDGES = 320000
D_PACK = D_FEAT // 2

NC = 2
NS = 16
FS = 8
PW = D_PACK // FS
NSHARD = NS // FS
E_PER_SC = N_EDGES // NC
E_PER_SHARD = E_PER_SC // NSHARD
BATCH_E = 4000
ROW_E = 2000
RPB = BATCH_E // ROW_E
NBATCH = E_PER_SHARD // BATCH_E
NPAIR = NBATCH // 2
NGROUP = ROW_E // 16
NROWS = NSHARD * NBATCH * RPB
ROWS_PER_TILE = NROWS // NS
LANES = 16
LO_MASK = 0xFFFF


def _sc_body(hpt_hbm, idx_hbm, rid_hbm, out_hbm,
             t0, t1, t2, t3, t4, t5, t6, t7,
             ib0, ib1, pb0, pb1, rid_v, stage_v, acc_sh,
             sem_i0, sem_i1, sem_a0, sem_a1):
    cid = lax.axis_index("c")
    sid = lax.axis_index("s")
    fs = sid % FS
    shard = sid // FS
    tabs = (t0, t1, t2, t3, t4, t5, t6, t7)

    for kk in range(PW):
        pltpu.sync_copy(hpt_hbm.at[fs, kk], tabs[kk])
    pltpu.sync_copy(rid_hbm, rid_v)

    def zrow(r, carry):
        def zcol(i, carry2):
            stage_v[r, pl.ds(i * LANES, LANES)] = jnp.zeros((LANES,), jnp.float32)
            return carry2
        lax.fori_loop(0, ROW_E // LANES, zcol, 0, unroll=False)
        return carry
    lax.fori_loop(0, ROWS_PER_TILE, zrow, 0, unroll=False)
    pltpu.sync_copy(stage_v, acc_sh.at[pl.ds(sid * ROWS_PER_TILE, ROWS_PER_TILE)])

    plsc.subcore_barrier()

    ibufs = ((ib0, sem_i0), (ib1, sem_i1))
    pbufs = ((pb0, sem_a0), (pb1, sem_a1))

    def issue_idx(b, k):
        ib, si = ibufs[k]
        pltpu.async_copy(idx_hbm.at[cid, shard, b], ib, si)

    def wait_idx(k):
        ib, si = ibufs[k]
        pltpu.make_async_copy(idx_hbm.at[0, 0, 0], ib, si).wait()

    def issue_add(b, k):
        pb, sa = pbufs[k]
        pltpu.async_copy(pb, acc_sh.at[rid_v.at[shard * NBATCH + b]], sa, add=True)

    def wait_add(k):
        pb, sa = pbufs[k]
        pltpu.make_async_copy(pb, acc_sh.at[rid_v.at[0]], sa).wait()

    def compute(k):
        ib = ibufs[k][0]
        pb = pbufs[k][0]

        @plsc.parallel_loop(0, NGROUP, step=1, unroll=5)
        def _(g):
            sw = ib[0, pl.ds(g * LANES, LANES)]
            dw = ib[1, pl.ds(g * LANES, LANES)]
            su0 = sw & LO_MASK
            su1 = lax.shift_right_logical(sw, 16)
            dv0 = dw & LO_MASK
            dv1 = lax.shift_right_logical(dw, 16)
            a00 = jnp.zeros((LANES,), jnp.float32)
            a01 = jnp.zeros((LANES,), jnp.float32)
            a10 = jnp.zeros((LANES,), jnp.float32)
            a11 = jnp.zeros((LANES,), jnp.float32)
            for kk in range(PW):
                pu0 = plsc.load_gather(tabs[kk], [su0])
                pv0 = plsc.load_gather(tabs[kk], [dv0])
                pu1 = plsc.load_gather(tabs[kk], [su1])
                pv1 = plsc.load_gather(tabs[kk], [dv1])
                a00 = a00 + (plsc.bitcast(lax.shift_left(pu0, 16), jnp.float32)
                             * plsc.bitcast(lax.shift_left(pv0, 16), jnp.float32))
                a01 = a01 + (plsc.bitcast(pu0, jnp.float32)
                             * plsc.bitcast(pv0, jnp.float32))
                a10 = a10 + (plsc.bitcast(lax.shift_left(pu1, 16), jnp.float32)
                             * plsc.bitcast(lax.shift_left(pv1, 16), jnp.float32))
                a11 = a11 + (plsc.bitcast(pu1, jnp.float32)
                             * plsc.bitcast(pv1, jnp.float32))
            pb[0, pl.ds(g * LANES, LANES)] = a00 + a01
            pb[1, pl.ds(g * LANES, LANES)] = a10 + a11

    issue_idx(0, 0)

    def pair_body(p, carry):
        b0 = 2 * p
        issue_idx(b0 + 1, 1)
        wait_idx(0)

        @pl.when(p >= 1)
        def _():
            wait_add(0)

        compute(0)
        issue_add(b0, 0)

        @pl.when(p <= NPAIR - 2)
        def _():
            issue_idx(b0 + 2, 0)

        wait_idx(1)

        @pl.when(p >= 1)
        def _():
            wait_add(1)

        compute(1)
        issue_add(b0 + 1, 1)
        return carry

    lax.fori_loop(0, NPAIR, pair_body, 0, unroll=False)
    wait_add(0)
    wait_add(1)

    plsc.subcore_barrier()

    pltpu.sync_copy(acc_sh.at[pl.ds(sid * ROWS_PER_TILE, ROWS_PER_TILE)], stage_v)
    pltpu.sync_copy(stage_v, out_hbm.at[cid, pl.ds(sid * ROWS_PER_TILE, ROWS_PER_TILE)])


@jax.jit
def kernel(h, edge_index):
    hp = lax.bitcast_convert_type(
        h.astype(jnp.bfloat16).reshape(N_NODES, D_PACK, 2), jnp.int32)
    hpt = hp.reshape(N_NODES, FS, PW).transpose(1, 2, 0)
    ei = edge_index.astype(jnp.int32)
    halves = jnp.stack([ei[0], ei[1]], axis=0).reshape(
        2, NC, NSHARD, NBATCH, RPB, ROW_E)
    packed = halves[:, :, :, :, 0, :] | (halves[:, :, :, :, 1, :] << 16)
    idx = jnp.moveaxis(packed, 0, 3)
    rowids = jnp.arange(NROWS, dtype=jnp.int32).reshape(NSHARD * NBATCH, RPB)
    mesh = plsc.VectorSubcoreMesh(core_axis_name="c", subcore_axis_name="s")
    run = pl.kernel(
        _sc_body,
        out_type=jax.ShapeDtypeStruct((NC, NROWS, ROW_E), jnp.float32),
        mesh=mesh,
        compiler_params=pltpu.CompilerParams(needs_layout_passes=False,
                                             use_tc_tiling_on_sc=False),
        scratch_types=(
            [pltpu.VMEM((N_NODES,), jnp.int32) for _ in range(PW)] +
            [
                pltpu.VMEM((2, ROW_E), jnp.int32),
                pltpu.VMEM((2, ROW_E), jnp.int32),
                pltpu.VMEM((RPB, ROW_E), jnp.float32),
                pltpu.VMEM((RPB, ROW_E), jnp.float32),
                pltpu.VMEM((NSHARD * NBATCH, RPB), jnp.int32),
                pltpu.VMEM((ROWS_PER_TILE, ROW_E), jnp.float32),
                pltpu.VMEM_SHARED((NROWS, ROW_E), jnp.float32),
                pltpu.SemaphoreType.DMA,
                pltpu.SemaphoreType.DMA,
                pltpu.SemaphoreType.DMA,
                pltpu.SemaphoreType.DMA,
            ]
        ),
    )
    out = run(hpt, idx, rowids)
    return out.reshape(N_EDGES, 1)
